# baseline (speedup 1.0000x reference)
import functools

import jax
import jax.numpy as jnp
from jax import lax
from jax.experimental import pallas as pl
from jax.experimental.pallas import tpu as pltpu
from jax.experimental.pallas import tpu_sc as plsc

NC = 2
NS = 16
NW = NC * NS
LANES = 16
CHUNK = 128

N_NODES = 50000
HALF = N_NODES // 2
GARB = HALF
ACC_ROWS = 25088
HID = 64

_MESH = plsc.VectorSubcoreMesh(
    core_axis_name="c", subcore_axis_name="s", num_cores=NC, num_subcores=NS)


def _subgathers(k):
  offs = []
  o = 0
  while o < k:
    offs.append((o, min(CHUNK, k - o)))
    o += CHUNK
  return offs


def _sc_gather(table, idx, kmac=1000):
  B = idx.shape[0]
  n, d = table.shape
  dt = table.dtype
  bw = B // NW
  nmac = bw // kmac
  assert B % NW == 0 and bw % kmac == 0 and kmac % 8 == 0 and nmac >= 2
  assert d == HID
  subs = _subgathers(kmac)
  scratch = [
      pltpu.VMEM((kmac,), jnp.int32), pltpu.VMEM((kmac,), jnp.int32),
      pltpu.VMEM((kmac, d), dt), pltpu.VMEM((kmac, d), dt),
      pltpu.SemaphoreType.DMA, pltpu.SemaphoreType.DMA,
      pltpu.SemaphoreType.DMA, pltpu.SemaphoreType.DMA,
      pltpu.SemaphoreType.DMA, pltpu.SemaphoreType.DMA,
  ]

  @functools.partial(
      pl.kernel,
      out_type=jax.ShapeDtypeStruct((B, d), dt),
      mesh=_MESH,
      scratch_types=scratch,
      compiler_params=pltpu.CompilerParams(use_tc_tiling_on_sc=False),
  )
  def k(table_hbm, idx_hbm, out_hbm, ix0, ix1, rw0, rw1,
        si0, si1, sg0, sg1, so0, so1):
    wid = lax.axis_index("s") * NC + lax.axis_index("c")
    base = wid * bw
    ix = (ix0, ix1)
    rw = (rw0, rw1)
    si = (si0, si1)
    sg = (sg0, sg1)
    so = (so0, so1)

    def start_idx(i, b):
      pltpu.async_copy(idx_hbm.at[pl.ds(base + i * kmac, kmac)], ix[b], si[b])

    def wait_idx(b):
      pltpu.make_async_copy(idx_hbm.at[pl.ds(base, kmac)], ix[b], si[b]).wait()

    def fire_gathers(b):
      for (o, ln) in subs:
        pltpu.async_copy(table_hbm.at[ix[b].at[pl.ds(o, ln)]],
                         rw[b].at[pl.ds(o, ln)], sg[b])

    def drain_gathers(b):
      pltpu.make_async_copy(out_hbm.at[pl.ds(base, kmac)],
                            rw[b], sg[b]).wait()

    def start_store(i, b):
      pltpu.async_copy(rw[b], out_hbm.at[pl.ds(base + i * kmac, kmac)],
                       so[b])

    def wait_store(b):
      pltpu.make_async_copy(rw[b], out_hbm.at[pl.ds(base, kmac)],
                            so[b]).wait()

    start_idx(0, 0)
    wait_idx(0)
    fire_gathers(0)
    if nmac > 1:
      start_idx(1, 1)

    def pair_body(mi, carry):
      for b in (0, 1):
        i = 2 * mi + b
        nb = 1 - b

        @pl.when(i < nmac)
        def _():
          @pl.when(i + 1 < nmac)
          def _():
            wait_idx(nb)

            @pl.when(i >= 1)
            def _():
              wait_store(nb)
            fire_gathers(nb)
          drain_gathers(b)

          @pl.when(i + 2 < nmac)
          def _():
            start_idx(i + 2, b)
          start_store(i, b)
      return carry

    lax.fori_loop(0, (nmac + 1) // 2, pair_body, 0)
    wait_store(0)
    wait_store(1)

  return k(table, idx)


GOUT_PAD = 1024


def _sc_segsum(vals, sidx, zeros, gather_table=None, gidx=None, kmac=80,
               gather_out_idx=None):
  B = sidx.shape[0]
  bt = B // NS
  nmac = bt // kmac
  assert B % NS == 0 and bt % kmac == 0 and kmac % 16 == 0 and nmac >= 2
  assert kmac <= CHUNK
  gathered = gather_table is not None
  gout = gather_out_idx is not None
  scratch = [
      pltpu.VMEM((kmac,), jnp.int32), pltpu.VMEM((kmac,), jnp.int32),
      pltpu.VMEM((kmac, HID), jnp.float32),
      pltpu.VMEM((kmac, HID), jnp.float32),
      pltpu.VMEM((1, kmac), jnp.int32),
      pltpu.SemaphoreType.DMA, pltpu.SemaphoreType.DMA,
      pltpu.SemaphoreType.DMA, pltpu.SemaphoreType.DMA,
      pltpu.SemaphoreType.DMA,
      pltpu.VMEM_SHARED((ACC_ROWS, HID), jnp.float32),
  ]
  if gathered:
    scratch += [pltpu.VMEM((kmac,), jnp.int32), pltpu.VMEM((kmac,), jnp.int32)]
  if gout:
    scratch += [
        pltpu.VMEM((kmac,), jnp.int32), pltpu.VMEM((kmac,), jnp.int32),
        pltpu.VMEM((kmac,), jnp.int32), pltpu.VMEM((kmac,), jnp.int32),
        pltpu.VMEM((1, kmac), jnp.int32), pltpu.VMEM((1, kmac), jnp.int32),
        pltpu.SemaphoreType.DMA, pltpu.SemaphoreType.DMA,
    ]

  ins = (gather_table, gidx, sidx, zeros) if gathered else (vals, sidx, zeros)
  if gout:
    ins = ins + (gather_out_idx,)
  out_sh = (B + GOUT_PAD, HID) if gout else (N_NODES, HID)

  @functools.partial(
      pl.kernel,
      out_type=jax.ShapeDtypeStruct(out_sh, jnp.float32),
      mesh=_MESH,
      scratch_types=scratch,
      compiler_params=pltpu.CompilerParams(use_tc_tiling_on_sc=False),
  )
  def k(*args):
    a = list(args)
    if gathered:
      table_hbm, gidx_hbm, sidx_hbm, zeros_hbm = a[:4]
      del a[:4]
      vals_hbm = None
    else:
      vals_hbm, sidx_hbm, zeros_hbm = a[:3]
      del a[:3]
      table_hbm = gidx_hbm = None
    src_hbm = a.pop(0) if gout else None
    out_hbm = a.pop(0)
    ix0, ix1, rw0, rw1, sbuf, si0, si1, sr0, sr1, ss, acc = a[:11]
    del a[:11]
    gx = (a.pop(0), a.pop(0)) if gathered else (None, None)
    if gout:
      sx = (a.pop(0), a.pop(0))
      lx = (a.pop(0), a.pop(0))
      ox = (a.pop(0), a.pop(0))
      so = (a.pop(0), a.pop(0))
    ix = (ix0, ix1)
    rw = (rw0, rw1)
    si = (si0, si1)
    sr = (sr0, sr1)
    c = lax.axis_index("c")
    s = lax.axis_index("s")
    nodebase = c * HALF

    zr = ACC_ROWS // NS
    pltpu.sync_copy(zeros_hbm.at[pl.ds(s * zr, zr)], acc.at[pl.ds(s * zr, zr)])
    plsc.subcore_barrier()

    def start_idx(i, b):
      off = s * bt + i * kmac
      pltpu.async_copy(sidx_hbm.at[pl.ds(off, kmac)], ix[b], si[b])
      if gathered:
        pltpu.async_copy(gidx_hbm.at[pl.ds(off, kmac)], gx[b], si[b])

    def wait_idx(b):
      pltpu.make_async_copy(sidx_hbm.at[pl.ds(s * bt, kmac)],
                            ix[b], si[b]).wait()
      if gathered:
        pltpu.make_async_copy(gidx_hbm.at[pl.ds(s * bt, kmac)],
                              gx[b], si[b]).wait()

    def fire_rows(i, b):
      if gathered:
        pltpu.async_copy(table_hbm.at[gx[b]], rw[b], sr[b])
      else:
        off = s * bt + i * kmac
        pltpu.async_copy(vals_hbm.at[pl.ds(off, kmac)], rw[b], sr[b])

    def wait_rows(b):
      pltpu.make_async_copy(zeros_hbm.at[pl.ds(0, kmac)], rw[b],
                            sr[b]).wait()

    def compute_redirect(b):
      garb = jnp.full((LANES,), GARB, jnp.int32)
      for v in range(kmac // LANES):
        w = ix[b][pl.ds(v * LANES, LANES)]
        rel = w - nodebase
        ok = (rel >= 0) & (rel < HALF)
        sbuf[0, pl.ds(v * LANES, LANES)] = jnp.where(ok, rel, garb)

    def fire_scatters(b):
      pltpu.async_copy(rw[b], acc.at[sbuf.at[0]], ss, add=True).wait()

    start_idx(0, 0)
    wait_idx(0)
    fire_rows(0, 0)
    if nmac > 1:
      start_idx(1, 1)

    def pair_body(mi, carry):
      for b in (0, 1):
        i = 2 * mi + b
        nb = 1 - b

        @pl.when(i < nmac)
        def _():
          compute_redirect(b)

          @pl.when(i + 1 < nmac)
          def _():
            wait_idx(nb)
            fire_rows(i + 1, nb)
          wait_rows(b)

          @pl.when(i + 2 < nmac)
          def _():
            start_idx(i + 2, b)
          fire_scatters(b)
      return carry

    lax.fori_loop(0, (nmac + 1) // 2, pair_body, 0)
    plsc.subcore_barrier()

    if not gout:
      pr = 1560
      rem = HALF - pr * NS
      pltpu.sync_copy(acc.at[pl.ds(s * pr, pr)],
                      out_hbm.at[pl.ds(nodebase + s * pr, pr)])
      @pl.when(s == NS - 1)
      def _():
        pltpu.sync_copy(acc.at[pl.ds(NS * pr, rem)],
                        out_hbm.at[pl.ds(NS * pr + nodebase, rem)])
      return

    def g_start_idx(i, b):
      pltpu.async_copy(src_hbm.at[pl.ds(s * bt + i * kmac, kmac)],
                       sx[b], si[b])

    def g_wait_idx(b):
      pltpu.make_async_copy(src_hbm.at[pl.ds(s * bt, kmac)],
                            sx[b], si[b]).wait()

    def g_compute(i, b):
      garb = jnp.full((LANES,), GARB, jnp.int32)
      base = s * bt + i * kmac
      for v in range(kmac // LANES):
        w = sx[b][pl.ds(v * LANES, LANES)]
        rel = w - nodebase
        ok = (rel >= 0) & (rel < HALF)
        lx[b][pl.ds(v * LANES, LANES)] = jnp.where(ok, rel, garb)
        pos = base + v * LANES + lax.iota(jnp.int32, LANES)
        gpos = B + ((s * LANES + lax.iota(jnp.int32, LANES))
                    + v * LANES) % GOUT_PAD
        ox[b][0, pl.ds(v * LANES, LANES)] = jnp.where(ok, pos, gpos)

    def g_fire_rows(b):
      pltpu.async_copy(acc.at[lx[b]], rw[b], sr[b])

    def g_wait_rows(b):
      pltpu.make_async_copy(zeros_hbm.at[pl.ds(0, kmac)], rw[b],
                            sr[b]).wait()

    def g_fire_out(b):
      pltpu.async_copy(rw[b], out_hbm.at[ox[b].at[0]], so[b])

    def g_wait_out(b):
      pltpu.make_async_copy(rw[b], out_hbm.at[pl.ds(0, kmac)], so[b]).wait()

    g_start_idx(0, 0)
    g_wait_idx(0)
    g_compute(0, 0)
    g_fire_rows(0)
    if nmac > 1:
      g_start_idx(1, 1)

    def g_pair_body(mi, carry):
      for b in (0, 1):
        i = 2 * mi + b
        nb = 1 - b

        @pl.when(i < nmac)
        def _():
          @pl.when(i + 1 < nmac)
          def _():
            g_wait_idx(nb)

            @pl.when(i >= 1)
            def _():
              g_wait_out(nb)
            g_compute(i + 1, nb)
            g_fire_rows(nb)
          g_wait_rows(b)

          @pl.when(i + 2 < nmac)
          def _():
            g_start_idx(i + 2, b)
          g_fire_out(b)
      return carry

    lax.fori_loop(0, (nmac + 1) // 2, g_pair_body, 0)
    g_wait_out(0)
    g_wait_out(1)

  return k(*ins)


ACC2_ROWS = 50048


def _sc_segsum_cols(vals, sidx, zeros2, kmac=80, gather_out_idx=None):
  B = sidx.shape[0]
  gout = gather_out_idx is not None
  hh = HID // NC
  bt = B // NS
  nmac = bt // kmac
  assert B % NS == 0 and bt % kmac == 0 and kmac % 16 == 0 and nmac >= 2
  assert kmac <= CHUNK
  scratch = [
      pltpu.VMEM((kmac,), jnp.int32), pltpu.VMEM((kmac,), jnp.int32),
      pltpu.VMEM((kmac, hh), jnp.float32), pltpu.VMEM((kmac, hh), jnp.float32),
      pltpu.VMEM((1, kmac), jnp.int32),
      pltpu.SemaphoreType.DMA, pltpu.SemaphoreType.DMA,
      pltpu.SemaphoreType.DMA, pltpu.SemaphoreType.DMA,
      pltpu.SemaphoreType.DMA,
      pltpu.VMEM_SHARED((ACC2_ROWS, hh), jnp.float32),
  ]
  if gout:
    scratch += [
        pltpu.VMEM((kmac,), jnp.int32), pltpu.VMEM((kmac,), jnp.int32),
        pltpu.SemaphoreType.DMA, pltpu.SemaphoreType.DMA,
    ]
  ins = (vals, sidx, zeros2) + ((gather_out_idx,) if gout else ())
  out_sh = (B, HID) if gout else (N_NODES, HID)

  @functools.partial(
      pl.kernel,
      out_type=jax.ShapeDtypeStruct(out_sh, jnp.float32),
      mesh=_MESH,
      scratch_types=scratch,
      compiler_params=pltpu.CompilerParams(use_tc_tiling_on_sc=False),
  )
  def k(*args):
    a = list(args)
    vals_hbm, sidx_hbm, zeros_hbm = a[:3]
    del a[:3]
    src_hbm = a.pop(0) if gout else None
    out_hbm = a.pop(0)
    ix0, ix1, rw0, rw1, sbuf, si0, si1, sr0, sr1, ss, acc = a[:11]
    del a[:11]
    if gout:
      sx = (a.pop(0), a.pop(0))
      so = (a.pop(0), a.pop(0))
    ix = (ix0, ix1)
    rw = (rw0, rw1)
    si = (si0, si1)
    sr = (sr0, sr1)
    c = lax.axis_index("c")
    s = lax.axis_index("s")
    colbase = c * hh

    zr = ACC2_ROWS // NS
    pltpu.sync_copy(zeros_hbm.at[pl.ds(s * zr, zr)], acc.at[pl.ds(s * zr, zr)])
    plsc.subcore_barrier()

    def start_idx(i, b):
      pltpu.async_copy(sidx_hbm.at[pl.ds(s * bt + i * kmac, kmac)],
                       ix[b], si[b])

    def wait_idx(b):
      pltpu.make_async_copy(sidx_hbm.at[pl.ds(s * bt, kmac)],
                            ix[b], si[b]).wait()

    def fire_rows(i, b):
      pltpu.async_copy(
          vals_hbm.at[pl.ds(s * bt + i * kmac, kmac), pl.ds(colbase, hh)],
          rw[b], sr[b])

    def wait_rows(b):
      pltpu.make_async_copy(
          vals_hbm.at[pl.ds(s * bt, kmac), pl.ds(colbase, hh)],
          rw[b], sr[b]).wait()

    def compute_redirect(b):
      garb = jnp.full((LANES,), N_NODES, jnp.int32)
      for v in range(kmac // LANES):
        w = ix[b][pl.ds(v * LANES, LANES)]
        ok = (w >= 0) & (w < N_NODES)
        sbuf[0, pl.ds(v * LANES, LANES)] = jnp.where(ok, w, garb)

    def fire_scatters(b):
      pltpu.async_copy(rw[b], acc.at[sbuf.at[0]], ss, add=True).wait()

    start_idx(0, 0)
    wait_idx(0)
    fire_rows(0, 0)
    if nmac > 1:
      start_idx(1, 1)

    def pair_body(mi, carry):
      for b in (0, 1):
        i = 2 * mi + b
        nb = 1 - b

        @pl.when(i < nmac)
        def _():
          compute_redirect(b)

          @pl.when(i + 1 < nmac)
          def _():
            wait_idx(nb)
            fire_rows(i + 1, nb)
          wait_rows(b)

          @pl.when(i + 2 < nmac)
          def _():
            start_idx(i + 2, b)
          fire_scatters(b)
      return carry

    lax.fori_loop(0, (nmac + 1) // 2, pair_body, 0)
    plsc.subcore_barrier()

    if not gout:
      pr = 3120
      rem = N_NODES - pr * NS
      pltpu.sync_copy(acc.at[pl.ds(s * pr, pr)],
                      out_hbm.at[pl.ds(s * pr, pr), pl.ds(colbase, hh)])
      @pl.when(s == NS - 1)
      def _():
        pltpu.sync_copy(acc.at[pl.ds(NS * pr, rem)],
                        out_hbm.at[pl.ds(NS * pr, rem), pl.ds(colbase, hh)])
      return

    def g_start_idx(i, b):
      pltpu.async_copy(src_hbm.at[pl.ds(s * bt + i * kmac, kmac)],
                       sx[b], si[b])

    def g_wait_idx(b):
      pltpu.make_async_copy(src_hbm.at[pl.ds(s * bt, kmac)],
                            sx[b], si[b]).wait()

    def g_fire_rows(b):
      pltpu.async_copy(acc.at[sx[b]], rw[b], sr[b])

    def g_wait_rows(b):
      pltpu.make_async_copy(zeros_hbm.at[pl.ds(0, kmac)], rw[b],
                            sr[b]).wait()

    def g_fire_out(i, b):
      pltpu.async_copy(
          rw[b],
          out_hbm.at[pl.ds(s * bt + i * kmac, kmac), pl.ds(colbase, hh)],
          so[b])

    def g_wait_out(b):
      pltpu.make_async_copy(
          rw[b], out_hbm.at[pl.ds(s * bt, kmac), pl.ds(colbase, hh)],
          so[b]).wait()

    g_start_idx(0, 0)
    g_wait_idx(0)
    g_fire_rows(0)
    if nmac > 1:
      g_start_idx(1, 1)

    def g_pair_body(mi, carry):
      for b in (0, 1):
        i = 2 * mi + b
        nb = 1 - b

        @pl.when(i < nmac)
        def _():
          @pl.when(i + 1 < nmac)
          def _():
            g_wait_idx(nb)

            @pl.when(i >= 1)
            def _():
              g_wait_out(nb)
            g_fire_rows(nb)
          g_wait_rows(b)

          @pl.when(i + 2 < nmac)
          def _():
            g_start_idx(i + 2, b)
          g_fire_out(i, b)
      return carry

    lax.fori_loop(0, (nmac + 1) // 2, g_pair_body, 0)
    g_wait_out(0)
    g_wait_out(1)

  return k(*ins)


def _tc_matmul(a, w):
  m, kdim = a.shape
  n = w.shape[1]
  r = 1600 if m % 1600 == 0 else 1000
  assert m % r == 0
  def body(a_ref, w_ref, o_ref):
    o_ref[...] = jnp.dot(a_ref[...], w_ref[...],
                         preferred_element_type=jnp.float32)
  return pl.pallas_call(
      body,
      grid=(m // r,),
      in_specs=[pl.BlockSpec((r, kdim), lambda i: (i, 0)),
                pl.BlockSpec((kdim, n), lambda i: (0, 0))],
      out_specs=pl.BlockSpec((r, n), lambda i: (i, 0)),
      out_shape=jax.ShapeDtypeStruct((m, n), jnp.float32),
  )(a, w)


def _tc_nodeprep(xpad, na, w1, wh):
  m = xpad.shape[0]
  r = 1000
  def body(x_ref, na_ref, w1_ref, wh_ref, xw_ref, nb_ref):
    xw = jnp.dot(x_ref[...], w1_ref[...], preferred_element_type=jnp.float32)
    xw_ref[...] = xw
    nb_ref[...] = xw + jnp.dot(na_ref[...], wh_ref[...],
                               preferred_element_type=jnp.float32)
  return pl.pallas_call(
      body,
      grid=(m // r,),
      in_specs=[pl.BlockSpec((r, HID), lambda i: (i, 0)),
                pl.BlockSpec((r, HID), lambda i: (i, 0)),
                pl.BlockSpec((HID, HID), lambda i: (0, 0)),
                pl.BlockSpec((HID, HID), lambda i: (0, 0))],
      out_specs=[pl.BlockSpec((r, HID), lambda i: (i, 0)),
                 pl.BlockSpec((r, HID), lambda i: (i, 0))],
      out_shape=[jax.ShapeDtypeStruct((m, HID), jnp.float32),
                 jax.ShapeDtypeStruct((m, HID), jnp.float32)],
  )(xpad, na, w1, wh)


def _tc_cast_bf16(x):
  m, d = x.shape
  r = 1000
  def body(x_ref, o_ref):
    o_ref[...] = x_ref[...].astype(jnp.bfloat16)
  return pl.pallas_call(
      body,
      grid=(m // r,),
      in_specs=[pl.BlockSpec((r, d), lambda i: (i, 0))],
      out_specs=pl.BlockSpec((r, d), lambda i: (i, 0)),
      out_shape=jax.ShapeDtypeStruct((m, d), jnp.bfloat16),
  )(x)


def _tc_combine(g1, g2, ew):
  m, w = g1.shape
  r = 1600
  def body(g1_ref, g2_ref, ew_ref, msg_ref, b_ref):
    e = ew_ref[...]
    msg_ref[...] = jnp.maximum(g1_ref[...].astype(jnp.float32) + e, 0.0)
    b_ref[...] = g2_ref[...].astype(jnp.float32) + e
  return pl.pallas_call(
      body,
      grid=(m // r,),
      in_specs=[pl.BlockSpec((r, w), lambda i: (i, 0))] * 3,
      out_specs=[pl.BlockSpec((r, w), lambda i: (i, 0))] * 2,
      out_shape=[jax.ShapeDtypeStruct((m, w), jnp.float32),
                 jax.ShapeDtypeStruct((m, w), jnp.float32)],
  )(g1, g2, ew)


def _tc_update(bmat, g, msg, whs):
  m, w = bmat.shape
  r = 1600
  def body(b_ref, g_ref, msg_ref, wh_ref, o_ref):
    mm = msg_ref[...]
    up = pltpu.roll(mm, r - 1, 0)
    dn = pltpu.roll(mm, 1, 0)
    even = (lax.broadcasted_iota(jnp.int32, (r, w), 0) % 2) == 0
    mrev = jnp.where(even, up, dn)
    d = g_ref[...].astype(jnp.float32) - mrev
    o_ref[...] = jnp.maximum(
        b_ref[...].astype(jnp.float32)
        + jnp.dot(d, wh_ref[...], preferred_element_type=jnp.float32), 0.0)
  return pl.pallas_call(
      body,
      grid=(m // r,),
      in_specs=[pl.BlockSpec((r, w), lambda i: (i, 0))] * 3
      + [pl.BlockSpec((w, w), lambda i: (0, 0))],
      out_specs=pl.BlockSpec((r, w), lambda i: (i, 0)),
      out_shape=jax.ShapeDtypeStruct((m, w), jnp.float32),
  )(bmat, g, msg, whs)


def _tc_final(xpad, m, na, gids, w1, w2, b2, n_graphs):
  n = xpad.shape[0]
  r = 1000
  grid = n // r

  def body(x_ref, m_ref, na_ref, gid_ref, w1_ref, w2_ref, b_ref, out_ref,
           acc_ref, cnt_ref):
    i = pl.program_id(0)

    @pl.when(i == 0)
    def _():
      acc_ref[...] = jnp.zeros_like(acc_ref)
      cnt_ref[...] = jnp.zeros_like(cnt_ref)

    h = jnp.maximum(
        jnp.dot(x_ref[...], w1_ref[...], preferred_element_type=jnp.float32)
        + jnp.dot(m_ref[...] + na_ref[...], w2_ref[...],
                  preferred_element_type=jnp.float32)
        + b_ref[...], 0.0)
    onehot = (gid_ref[...] == lax.broadcasted_iota(
        jnp.int32, (r, n_graphs), 1)).astype(jnp.float32)
    acc_ref[...] += lax.dot_general(
        onehot, h, (((0,), (0,)), ((), ())),
        preferred_element_type=jnp.float32)
    cnt_ref[...] += lax.dot_general(
        onehot, jnp.ones((r, 8), jnp.float32), (((0,), (0,)), ((), ())),
        preferred_element_type=jnp.float32)

    @pl.when(i == grid - 1)
    def _():
      out_ref[...] = acc_ref[...] / jnp.maximum(cnt_ref[:, 0:1], 1.0)

  return pl.pallas_call(
      body,
      grid=(grid,),
      in_specs=[pl.BlockSpec((r, HID), lambda i: (i, 0)),
                pl.BlockSpec((r, HID), lambda i: (i, 0)),
                pl.BlockSpec((r, HID), lambda i: (i, 0)),
                pl.BlockSpec((r, 1), lambda i: (i, 0)),
                pl.BlockSpec((HID, HID), lambda i: (0, 0)),
                pl.BlockSpec((HID, HID), lambda i: (0, 0)),
                pl.BlockSpec((1, HID), lambda i: (0, 0))],
      out_specs=pl.BlockSpec((n_graphs, HID), lambda i: (0, 0)),
      out_shape=jax.ShapeDtypeStruct((n_graphs, HID), jnp.float32),
      scratch_shapes=[pltpu.VMEM((n_graphs, HID), jnp.float32),
                      pltpu.VMEM((n_graphs, 8), jnp.float32)],
  )(xpad, m, na, gids, w1, w2, b2)


def kernel(x, edge_attr, edge_index, tree_edge_m, tree_mess_src_eids,
           tree_mess_tgt_nodes, graph_ids, W_i, W_h, W_o, b_o):
  e = edge_attr.shape[0]
  n = x.shape[0]
  afd = x.shape[1]
  n_graphs = 128
  depth = 3

  src32 = edge_index[0].astype(jnp.int32)
  dst32 = edge_index[1].astype(jnp.int32)

  nt = tree_mess_src_eids.shape[0]
  tree_k = 80
  quant = NS * tree_k * 2
  ntp = ((nt + quant - 1) // quant) * quant
  eids32 = jnp.pad(tree_mess_src_eids.astype(jnp.int32), (0, ntp - nt))
  tgt32 = jnp.pad(tree_mess_tgt_nodes.astype(jnp.int32), (0, ntp - nt),
                  constant_values=-1)

  gids = graph_ids.astype(jnp.int32).reshape(n, 1)
  xpad = jnp.pad(x, ((0, 0), (0, HID - afd)))
  ea8 = jnp.pad(edge_attr, ((0, 0), (0, 3)))
  w_i1 = jnp.pad(W_i[:afd], ((0, HID - afd), (0, 0)))
  w_i2 = jnp.pad(W_i[afd:], ((0, 3), (0, 0)))
  w_o1 = jnp.pad(W_o[:afd], ((0, HID - afd), (0, 0)))
  w_o2 = W_o[afd:]
  b2 = b_o.reshape(1, HID)
  zeros_acc = jnp.zeros((ACC_ROWS, HID), jnp.float32)
  zeros2 = jnp.zeros((ACC2_ROWS, HID // NC), jnp.float32)

  node_alpha = _sc_segsum(None, tgt32, zeros_acc,
                          gather_table=tree_edge_m, gidx=eids32, kmac=tree_k)
  ew = _tc_matmul(ea8, w_i2)
  xw, node_b = _tc_nodeprep(xpad, node_alpha, w_i1, W_h)
  g1 = _sc_gather(xw, src32)
  g2 = _sc_gather(node_b, src32)
  msg, bmat = _tc_combine(g1, g2, ew)
  for _ in range(depth - 1):
    g = _sc_segsum_cols(msg, dst32, zeros2, gather_out_idx=src32)
    msg = _tc_update(bmat, g, msg, W_h)
  m = _sc_segsum_cols(msg, dst32, zeros2)
  return _tc_final(xpad, m, node_alpha, gids, w_o1, w_o2, b2, n_graphs)


# speedup vs baseline: 1.0375x; 1.0375x over previous
import functools

import jax
import jax.numpy as jnp
from jax import lax
from jax.experimental import pallas as pl
from jax.experimental.pallas import tpu as pltpu
from jax.experimental.pallas import tpu_sc as plsc

NC = 2
NS = 16
NW = NC * NS
LANES = 16
CHUNK = 128

N_NODES = 50000
HALF = N_NODES // 2
GARB = HALF
ACC_ROWS = 25088
HID = 64

_MESH = plsc.VectorSubcoreMesh(
    core_axis_name="c", subcore_axis_name="s", num_cores=NC, num_subcores=NS)


def _subgathers(k):
  offs = []
  o = 0
  while o < k:
    offs.append((o, min(CHUNK, k - o)))
    o += CHUNK
  return offs


def _sc_gather(table, idx, kmac=1000):
  B = idx.shape[0]
  n, d = table.shape
  dt = table.dtype
  bw = B // NW
  nmac = bw // kmac
  assert B % NW == 0 and bw % kmac == 0 and kmac % 8 == 0 and nmac >= 2
  assert d == HID
  subs = _subgathers(kmac)
  scratch = [
      pltpu.VMEM((kmac,), jnp.int32), pltpu.VMEM((kmac,), jnp.int32),
      pltpu.VMEM((kmac, d), dt), pltpu.VMEM((kmac, d), dt),
      pltpu.SemaphoreType.DMA, pltpu.SemaphoreType.DMA,
      pltpu.SemaphoreType.DMA, pltpu.SemaphoreType.DMA,
      pltpu.SemaphoreType.DMA, pltpu.SemaphoreType.DMA,
  ]

  @functools.partial(
      pl.kernel,
      out_type=jax.ShapeDtypeStruct((B, d), dt),
      mesh=_MESH,
      scratch_types=scratch,
      compiler_params=pltpu.CompilerParams(use_tc_tiling_on_sc=False),
  )
  def k(table_hbm, idx_hbm, out_hbm, ix0, ix1, rw0, rw1,
        si0, si1, sg0, sg1, so0, so1):
    wid = lax.axis_index("s") * NC + lax.axis_index("c")
    base = wid * bw
    ix = (ix0, ix1)
    rw = (rw0, rw1)
    si = (si0, si1)
    sg = (sg0, sg1)
    so = (so0, so1)

    def start_idx(i, b):
      pltpu.async_copy(idx_hbm.at[pl.ds(base + i * kmac, kmac)], ix[b], si[b])

    def wait_idx(b):
      pltpu.make_async_copy(idx_hbm.at[pl.ds(base, kmac)], ix[b], si[b]).wait()

    def fire_gathers(b):
      for (o, ln) in subs:
        pltpu.async_copy(table_hbm.at[ix[b].at[pl.ds(o, ln)]],
                         rw[b].at[pl.ds(o, ln)], sg[b])

    def drain_gathers(b):
      pltpu.make_async_copy(out_hbm.at[pl.ds(base, kmac)],
                            rw[b], sg[b]).wait()

    def start_store(i, b):
      pltpu.async_copy(rw[b], out_hbm.at[pl.ds(base + i * kmac, kmac)],
                       so[b])

    def wait_store(b):
      pltpu.make_async_copy(rw[b], out_hbm.at[pl.ds(base, kmac)],
                            so[b]).wait()

    start_idx(0, 0)
    wait_idx(0)
    fire_gathers(0)
    if nmac > 1:
      start_idx(1, 1)

    def pair_body(mi, carry):
      for b in (0, 1):
        i = 2 * mi + b
        nb = 1 - b

        @pl.when(i < nmac)
        def _():
          @pl.when(i + 1 < nmac)
          def _():
            wait_idx(nb)

            @pl.when(i >= 1)
            def _():
              wait_store(nb)
            fire_gathers(nb)
          drain_gathers(b)

          @pl.when(i + 2 < nmac)
          def _():
            start_idx(i + 2, b)
          start_store(i, b)
      return carry

    lax.fori_loop(0, (nmac + 1) // 2, pair_body, 0)
    wait_store(0)
    wait_store(1)

  return k(table, idx)


GOUT_PAD = 1024


def _sc_segsum(vals, sidx, zeros, gather_table=None, gidx=None, kmac=80,
               gather_out_idx=None):
  B = sidx.shape[0]
  bt = B // NS
  nmac = bt // kmac
  assert B % NS == 0 and bt % kmac == 0 and kmac % 16 == 0 and nmac >= 2
  assert kmac <= CHUNK
  gathered = gather_table is not None
  gout = gather_out_idx is not None
  scratch = [
      pltpu.VMEM((kmac,), jnp.int32), pltpu.VMEM((kmac,), jnp.int32),
      pltpu.VMEM((kmac, HID), jnp.float32),
      pltpu.VMEM((kmac, HID), jnp.float32),
      pltpu.VMEM((1, kmac), jnp.int32),
      pltpu.SemaphoreType.DMA, pltpu.SemaphoreType.DMA,
      pltpu.SemaphoreType.DMA, pltpu.SemaphoreType.DMA,
      pltpu.SemaphoreType.DMA,
      pltpu.VMEM_SHARED((ACC_ROWS, HID), jnp.float32),
  ]
  if gathered:
    scratch += [pltpu.VMEM((kmac,), jnp.int32), pltpu.VMEM((kmac,), jnp.int32)]
  if gout:
    scratch += [
        pltpu.VMEM((kmac,), jnp.int32), pltpu.VMEM((kmac,), jnp.int32),
        pltpu.VMEM((kmac,), jnp.int32), pltpu.VMEM((kmac,), jnp.int32),
        pltpu.VMEM((1, kmac), jnp.int32), pltpu.VMEM((1, kmac), jnp.int32),
        pltpu.SemaphoreType.DMA, pltpu.SemaphoreType.DMA,
    ]

  ins = (gather_table, gidx, sidx, zeros) if gathered else (vals, sidx, zeros)
  if gout:
    ins = ins + (gather_out_idx,)
  out_sh = (B + GOUT_PAD, HID) if gout else (N_NODES, HID)

  @functools.partial(
      pl.kernel,
      out_type=jax.ShapeDtypeStruct(out_sh, jnp.float32),
      mesh=_MESH,
      scratch_types=scratch,
      compiler_params=pltpu.CompilerParams(use_tc_tiling_on_sc=False),
  )
  def k(*args):
    a = list(args)
    if gathered:
      table_hbm, gidx_hbm, sidx_hbm, zeros_hbm = a[:4]
      del a[:4]
      vals_hbm = None
    else:
      vals_hbm, sidx_hbm, zeros_hbm = a[:3]
      del a[:3]
      table_hbm = gidx_hbm = None
    src_hbm = a.pop(0) if gout else None
    out_hbm = a.pop(0)
    ix0, ix1, rw0, rw1, sbuf, si0, si1, sr0, sr1, ss, acc = a[:11]
    del a[:11]
    gx = (a.pop(0), a.pop(0)) if gathered else (None, None)
    if gout:
      sx = (a.pop(0), a.pop(0))
      lx = (a.pop(0), a.pop(0))
      ox = (a.pop(0), a.pop(0))
      so = (a.pop(0), a.pop(0))
    ix = (ix0, ix1)
    rw = (rw0, rw1)
    si = (si0, si1)
    sr = (sr0, sr1)
    c = lax.axis_index("c")
    s = lax.axis_index("s")
    nodebase = c * HALF

    zr = ACC_ROWS // NS
    pltpu.sync_copy(zeros_hbm.at[pl.ds(s * zr, zr)], acc.at[pl.ds(s * zr, zr)])
    plsc.subcore_barrier()

    def start_idx(i, b):
      off = s * bt + i * kmac
      pltpu.async_copy(sidx_hbm.at[pl.ds(off, kmac)], ix[b], si[b])
      if gathered:
        pltpu.async_copy(gidx_hbm.at[pl.ds(off, kmac)], gx[b], si[b])

    def wait_idx(b):
      pltpu.make_async_copy(sidx_hbm.at[pl.ds(s * bt, kmac)],
                            ix[b], si[b]).wait()
      if gathered:
        pltpu.make_async_copy(gidx_hbm.at[pl.ds(s * bt, kmac)],
                              gx[b], si[b]).wait()

    def fire_rows(i, b):
      if gathered:
        pltpu.async_copy(table_hbm.at[gx[b]], rw[b], sr[b])
      else:
        off = s * bt + i * kmac
        pltpu.async_copy(vals_hbm.at[pl.ds(off, kmac)], rw[b], sr[b])

    def wait_rows(b):
      pltpu.make_async_copy(zeros_hbm.at[pl.ds(0, kmac)], rw[b],
                            sr[b]).wait()

    def compute_redirect(b):
      garb = jnp.full((LANES,), GARB, jnp.int32)
      for v in range(kmac // LANES):
        w = ix[b][pl.ds(v * LANES, LANES)]
        rel = w - nodebase
        ok = (rel >= 0) & (rel < HALF)
        sbuf[0, pl.ds(v * LANES, LANES)] = jnp.where(ok, rel, garb)

    def fire_scatters(b):
      pltpu.async_copy(rw[b], acc.at[sbuf.at[0]], ss, add=True).wait()

    start_idx(0, 0)
    wait_idx(0)
    fire_rows(0, 0)
    if nmac > 1:
      start_idx(1, 1)

    def pair_body(mi, carry):
      for b in (0, 1):
        i = 2 * mi + b
        nb = 1 - b

        @pl.when(i < nmac)
        def _():
          compute_redirect(b)

          @pl.when(i + 1 < nmac)
          def _():
            wait_idx(nb)
            fire_rows(i + 1, nb)
          wait_rows(b)

          @pl.when(i + 2 < nmac)
          def _():
            start_idx(i + 2, b)
          fire_scatters(b)
      return carry

    lax.fori_loop(0, (nmac + 1) // 2, pair_body, 0)
    plsc.subcore_barrier()

    if not gout:
      pr = 1560
      rem = HALF - pr * NS
      pltpu.sync_copy(acc.at[pl.ds(s * pr, pr)],
                      out_hbm.at[pl.ds(nodebase + s * pr, pr)])
      @pl.when(s == NS - 1)
      def _():
        pltpu.sync_copy(acc.at[pl.ds(NS * pr, rem)],
                        out_hbm.at[pl.ds(NS * pr + nodebase, rem)])
      return

    def g_start_idx(i, b):
      pltpu.async_copy(src_hbm.at[pl.ds(s * bt + i * kmac, kmac)],
                       sx[b], si[b])

    def g_wait_idx(b):
      pltpu.make_async_copy(src_hbm.at[pl.ds(s * bt, kmac)],
                            sx[b], si[b]).wait()

    def g_compute(i, b):
      garb = jnp.full((LANES,), GARB, jnp.int32)
      base = s * bt + i * kmac
      for v in range(kmac // LANES):
        w = sx[b][pl.ds(v * LANES, LANES)]
        rel = w - nodebase
        ok = (rel >= 0) & (rel < HALF)
        lx[b][pl.ds(v * LANES, LANES)] = jnp.where(ok, rel, garb)
        pos = base + v * LANES + lax.iota(jnp.int32, LANES)
        gpos = B + ((s * LANES + lax.iota(jnp.int32, LANES))
                    + v * LANES) % GOUT_PAD
        ox[b][0, pl.ds(v * LANES, LANES)] = jnp.where(ok, pos, gpos)

    def g_fire_rows(b):
      pltpu.async_copy(acc.at[lx[b]], rw[b], sr[b])

    def g_wait_rows(b):
      pltpu.make_async_copy(zeros_hbm.at[pl.ds(0, kmac)], rw[b],
                            sr[b]).wait()

    def g_fire_out(b):
      pltpu.async_copy(rw[b], out_hbm.at[ox[b].at[0]], so[b])

    def g_wait_out(b):
      pltpu.make_async_copy(rw[b], out_hbm.at[pl.ds(0, kmac)], so[b]).wait()

    g_start_idx(0, 0)
    g_wait_idx(0)
    g_compute(0, 0)
    g_fire_rows(0)
    if nmac > 1:
      g_start_idx(1, 1)

    def g_pair_body(mi, carry):
      for b in (0, 1):
        i = 2 * mi + b
        nb = 1 - b

        @pl.when(i < nmac)
        def _():
          @pl.when(i + 1 < nmac)
          def _():
            g_wait_idx(nb)

            @pl.when(i >= 1)
            def _():
              g_wait_out(nb)
            g_compute(i + 1, nb)
            g_fire_rows(nb)
          g_wait_rows(b)

          @pl.when(i + 2 < nmac)
          def _():
            g_start_idx(i + 2, b)
          g_fire_out(b)
      return carry

    lax.fori_loop(0, (nmac + 1) // 2, g_pair_body, 0)
    g_wait_out(0)
    g_wait_out(1)

  return k(*ins)


ACC2_ROWS = 50048


def _sc_segsum_cols(vals, sidx, zeros2, kmac=80, gather_out_idx=None):
  B = sidx.shape[0]
  gout = gather_out_idx is not None
  hh = HID // NC
  bt = B // NS
  nmac = bt // kmac
  assert B % NS == 0 and bt % kmac == 0 and kmac % 16 == 0 and nmac >= 2
  assert kmac <= CHUNK
  scratch = [
      pltpu.VMEM((kmac,), jnp.int32), pltpu.VMEM((kmac,), jnp.int32),
      pltpu.VMEM((kmac, hh), jnp.float32), pltpu.VMEM((kmac, hh), jnp.float32),
      pltpu.VMEM((1, kmac), jnp.int32),
      pltpu.SemaphoreType.DMA, pltpu.SemaphoreType.DMA,
      pltpu.SemaphoreType.DMA, pltpu.SemaphoreType.DMA,
      pltpu.SemaphoreType.DMA,
      pltpu.VMEM_SHARED((ACC2_ROWS, hh), jnp.float32),
  ]
  if gout:
    scratch += [
        pltpu.VMEM((kmac,), jnp.int32), pltpu.VMEM((kmac,), jnp.int32),
        pltpu.SemaphoreType.DMA, pltpu.SemaphoreType.DMA,
    ]
  ins = (vals, sidx, zeros2) + ((gather_out_idx,) if gout else ())
  out_sh = (B, HID) if gout else (N_NODES, HID)

  @functools.partial(
      pl.kernel,
      out_type=jax.ShapeDtypeStruct(out_sh, jnp.float32),
      mesh=_MESH,
      scratch_types=scratch,
      compiler_params=pltpu.CompilerParams(use_tc_tiling_on_sc=False),
  )
  def k(*args):
    a = list(args)
    vals_hbm, sidx_hbm, zeros_hbm = a[:3]
    del a[:3]
    src_hbm = a.pop(0) if gout else None
    out_hbm = a.pop(0)
    ix0, ix1, rw0, rw1, sbuf, si0, si1, sr0, sr1, ss, acc = a[:11]
    del a[:11]
    if gout:
      sx = (a.pop(0), a.pop(0))
      so = (a.pop(0), a.pop(0))
    ix = (ix0, ix1)
    rw = (rw0, rw1)
    si = (si0, si1)
    sr = (sr0, sr1)
    c = lax.axis_index("c")
    s = lax.axis_index("s")
    colbase = c * hh

    zr = ACC2_ROWS // NS
    pltpu.sync_copy(zeros_hbm.at[pl.ds(s * zr, zr)], acc.at[pl.ds(s * zr, zr)])
    plsc.subcore_barrier()

    def start_idx(i, b):
      pltpu.async_copy(sidx_hbm.at[pl.ds(s * bt + i * kmac, kmac)],
                       ix[b], si[b])

    def wait_idx(b):
      pltpu.make_async_copy(sidx_hbm.at[pl.ds(s * bt, kmac)],
                            ix[b], si[b]).wait()

    def fire_rows(i, b):
      pltpu.async_copy(
          vals_hbm.at[pl.ds(s * bt + i * kmac, kmac), pl.ds(colbase, hh)],
          rw[b], sr[b])

    def wait_rows(b):
      pltpu.make_async_copy(
          vals_hbm.at[pl.ds(s * bt, kmac), pl.ds(colbase, hh)],
          rw[b], sr[b]).wait()

    def compute_redirect(b):
      garb = jnp.full((LANES,), N_NODES, jnp.int32)
      for v in range(kmac // LANES):
        w = ix[b][pl.ds(v * LANES, LANES)]
        ok = (w >= 0) & (w < N_NODES)
        sbuf[0, pl.ds(v * LANES, LANES)] = jnp.where(ok, w, garb)

    def fire_scatters(b):
      pltpu.async_copy(rw[b], acc.at[sbuf.at[0]], ss, add=True).wait()

    start_idx(0, 0)
    wait_idx(0)
    fire_rows(0, 0)
    if nmac > 1:
      start_idx(1, 1)

    def pair_body(mi, carry):
      for b in (0, 1):
        i = 2 * mi + b
        nb = 1 - b

        @pl.when(i < nmac)
        def _():
          compute_redirect(b)

          @pl.when(i + 1 < nmac)
          def _():
            wait_idx(nb)
            fire_rows(i + 1, nb)
          wait_rows(b)

          @pl.when(i + 2 < nmac)
          def _():
            start_idx(i + 2, b)
          fire_scatters(b)
      return carry

    lax.fori_loop(0, (nmac + 1) // 2, pair_body, 0)
    plsc.subcore_barrier()

    if not gout:
      pr = 3120
      rem = N_NODES - pr * NS
      pltpu.sync_copy(acc.at[pl.ds(s * pr, pr)],
                      out_hbm.at[pl.ds(s * pr, pr), pl.ds(colbase, hh)])
      @pl.when(s == NS - 1)
      def _():
        pltpu.sync_copy(acc.at[pl.ds(NS * pr, rem)],
                        out_hbm.at[pl.ds(NS * pr, rem), pl.ds(colbase, hh)])
      return

    def g_start_idx(i, b):
      pltpu.async_copy(src_hbm.at[pl.ds(s * bt + i * kmac, kmac)],
                       sx[b], si[b])

    def g_wait_idx(b):
      pltpu.make_async_copy(src_hbm.at[pl.ds(s * bt, kmac)],
                            sx[b], si[b]).wait()

    def g_fire_rows(b):
      pltpu.async_copy(acc.at[sx[b]], rw[b], sr[b])

    def g_wait_rows(b):
      pltpu.make_async_copy(zeros_hbm.at[pl.ds(0, kmac)], rw[b],
                            sr[b]).wait()

    def g_fire_out(i, b):
      pltpu.async_copy(
          rw[b],
          out_hbm.at[pl.ds(s * bt + i * kmac, kmac), pl.ds(colbase, hh)],
          so[b])

    def g_wait_out(b):
      pltpu.make_async_copy(
          rw[b], out_hbm.at[pl.ds(s * bt, kmac), pl.ds(colbase, hh)],
          so[b]).wait()

    g_start_idx(0, 0)
    g_wait_idx(0)
    g_fire_rows(0)
    if nmac > 1:
      g_start_idx(1, 1)

    def g_pair_body(mi, carry):
      for b in (0, 1):
        i = 2 * mi + b
        nb = 1 - b

        @pl.when(i < nmac)
        def _():
          @pl.when(i + 1 < nmac)
          def _():
            g_wait_idx(nb)

            @pl.when(i >= 1)
            def _():
              g_wait_out(nb)
            g_fire_rows(nb)
          g_wait_rows(b)

          @pl.when(i + 2 < nmac)
          def _():
            g_start_idx(i + 2, b)
          g_fire_out(i, b)
      return carry

    lax.fori_loop(0, (nmac + 1) // 2, g_pair_body, 0)
    g_wait_out(0)
    g_wait_out(1)

  return k(*ins)


def _tc_matmul(a, w):
  m, kdim = a.shape
  n = w.shape[1]
  r = 1600 if m % 1600 == 0 else 1000
  assert m % r == 0
  def body(a_ref, w_ref, o_ref):
    o_ref[...] = jnp.dot(a_ref[...], w_ref[...],
                         preferred_element_type=jnp.float32)
  return pl.pallas_call(
      body,
      grid=(m // r,),
      in_specs=[pl.BlockSpec((r, kdim), lambda i: (i, 0)),
                pl.BlockSpec((kdim, n), lambda i: (0, 0))],
      out_specs=pl.BlockSpec((r, n), lambda i: (i, 0)),
      out_shape=jax.ShapeDtypeStruct((m, n), jnp.float32),
  )(a, w)


def _tc_nodeprep(xpad, na, w1, wh):
  m = xpad.shape[0]
  r = 1000
  def body(x_ref, na_ref, w1_ref, wh_ref, xw_ref, nb_ref):
    xw = jnp.dot(x_ref[...], w1_ref[...], preferred_element_type=jnp.float32)
    xw_ref[...] = xw
    nb_ref[...] = xw + jnp.dot(na_ref[...], wh_ref[...],
                               preferred_element_type=jnp.float32)
  return pl.pallas_call(
      body,
      grid=(m // r,),
      in_specs=[pl.BlockSpec((r, HID), lambda i: (i, 0)),
                pl.BlockSpec((r, HID), lambda i: (i, 0)),
                pl.BlockSpec((HID, HID), lambda i: (0, 0)),
                pl.BlockSpec((HID, HID), lambda i: (0, 0))],
      out_specs=[pl.BlockSpec((r, HID), lambda i: (i, 0)),
                 pl.BlockSpec((r, HID), lambda i: (i, 0))],
      out_shape=[jax.ShapeDtypeStruct((m, HID), jnp.float32),
                 jax.ShapeDtypeStruct((m, HID), jnp.float32)],
  )(xpad, na, w1, wh)


def _tc_cast_bf16(x):
  m, d = x.shape
  r = 1000
  def body(x_ref, o_ref):
    o_ref[...] = x_ref[...].astype(jnp.bfloat16)
  return pl.pallas_call(
      body,
      grid=(m // r,),
      in_specs=[pl.BlockSpec((r, d), lambda i: (i, 0))],
      out_specs=pl.BlockSpec((r, d), lambda i: (i, 0)),
      out_shape=jax.ShapeDtypeStruct((m, d), jnp.bfloat16),
  )(x)


def _tc_combine(g1, g2, ew):
  m, w = g1.shape
  r = 1600
  def body(g1_ref, g2_ref, ew_ref, msg_ref, b_ref):
    e = ew_ref[...]
    msg_ref[...] = jnp.maximum(g1_ref[...].astype(jnp.float32) + e, 0.0)
    b_ref[...] = g2_ref[...].astype(jnp.float32) + e
  return pl.pallas_call(
      body,
      grid=(m // r,),
      in_specs=[pl.BlockSpec((r, w), lambda i: (i, 0))] * 3,
      out_specs=[pl.BlockSpec((r, w), lambda i: (i, 0))] * 2,
      out_shape=[jax.ShapeDtypeStruct((m, w), jnp.float32),
                 jax.ShapeDtypeStruct((m, w), jnp.float32)],
  )(g1, g2, ew)


def _tc_update(bmat, g, msg, whs):
  m, w = bmat.shape
  r = 1600
  def body(b_ref, g_ref, msg_ref, wh_ref, o_ref):
    mm = msg_ref[...]
    up = pltpu.roll(mm, r - 1, 0)
    dn = pltpu.roll(mm, 1, 0)
    even = (lax.broadcasted_iota(jnp.int32, (r, w), 0) % 2) == 0
    mrev = jnp.where(even, up, dn)
    d = g_ref[...].astype(jnp.float32) - mrev
    o_ref[...] = jnp.maximum(
        b_ref[...].astype(jnp.float32)
        + jnp.dot(d, wh_ref[...], preferred_element_type=jnp.float32), 0.0)
  return pl.pallas_call(
      body,
      grid=(m // r,),
      in_specs=[pl.BlockSpec((r, w), lambda i: (i, 0))] * 3
      + [pl.BlockSpec((w, w), lambda i: (0, 0))],
      out_specs=pl.BlockSpec((r, w), lambda i: (i, 0)),
      out_shape=jax.ShapeDtypeStruct((m, w), jnp.float32),
  )(bmat, g, msg, whs)


def _tc_final(xpad, m, na, gids, w1, w2, b2, n_graphs):
  n = xpad.shape[0]
  r = 1000
  grid = n // r

  def body(x_ref, m_ref, na_ref, gid_ref, w1_ref, w2_ref, b_ref, out_ref,
           acc_ref, cnt_ref):
    i = pl.program_id(0)

    @pl.when(i == 0)
    def _():
      acc_ref[...] = jnp.zeros_like(acc_ref)
      cnt_ref[...] = jnp.zeros_like(cnt_ref)

    h = jnp.maximum(
        jnp.dot(x_ref[...], w1_ref[...], preferred_element_type=jnp.float32)
        + jnp.dot(m_ref[...] + na_ref[...], w2_ref[...],
                  preferred_element_type=jnp.float32)
        + b_ref[...], 0.0)
    onehot = (gid_ref[...] == lax.broadcasted_iota(
        jnp.int32, (r, n_graphs), 1)).astype(jnp.float32)
    acc_ref[...] += lax.dot_general(
        onehot, h, (((0,), (0,)), ((), ())),
        preferred_element_type=jnp.float32)
    cnt_ref[...] += lax.dot_general(
        onehot, jnp.ones((r, 8), jnp.float32), (((0,), (0,)), ((), ())),
        preferred_element_type=jnp.float32)

    @pl.when(i == grid - 1)
    def _():
      out_ref[...] = acc_ref[...] / jnp.maximum(cnt_ref[:, 0:1], 1.0)

  return pl.pallas_call(
      body,
      grid=(grid,),
      in_specs=[pl.BlockSpec((r, HID), lambda i: (i, 0)),
                pl.BlockSpec((r, HID), lambda i: (i, 0)),
                pl.BlockSpec((r, HID), lambda i: (i, 0)),
                pl.BlockSpec((r, 1), lambda i: (i, 0)),
                pl.BlockSpec((HID, HID), lambda i: (0, 0)),
                pl.BlockSpec((HID, HID), lambda i: (0, 0)),
                pl.BlockSpec((1, HID), lambda i: (0, 0))],
      out_specs=pl.BlockSpec((n_graphs, HID), lambda i: (0, 0)),
      out_shape=jax.ShapeDtypeStruct((n_graphs, HID), jnp.float32),
      scratch_shapes=[pltpu.VMEM((n_graphs, HID), jnp.float32),
                      pltpu.VMEM((n_graphs, 8), jnp.float32)],
  )(xpad, m, na, gids, w1, w2, b2)


def kernel(x, edge_attr, edge_index, tree_edge_m, tree_mess_src_eids,
           tree_mess_tgt_nodes, graph_ids, W_i, W_h, W_o, b_o):
  e = edge_attr.shape[0]
  n = x.shape[0]
  afd = x.shape[1]
  n_graphs = 128
  depth = 3

  src32 = edge_index[0].astype(jnp.int32)
  dst32 = edge_index[1].astype(jnp.int32)

  nt = tree_mess_src_eids.shape[0]
  tree_k = 80
  quant = NS * tree_k * 2
  ntp = ((nt + quant - 1) // quant) * quant
  eids32 = jnp.pad(tree_mess_src_eids.astype(jnp.int32), (0, ntp - nt))
  tgt32 = jnp.pad(tree_mess_tgt_nodes.astype(jnp.int32), (0, ntp - nt),
                  constant_values=-1)

  gids = graph_ids.astype(jnp.int32).reshape(n, 1)
  xpad = jnp.pad(x, ((0, 0), (0, HID - afd)))
  ea8 = jnp.pad(edge_attr, ((0, 0), (0, 3)))
  w_i1 = jnp.pad(W_i[:afd], ((0, HID - afd), (0, 0)))
  w_i2 = jnp.pad(W_i[afd:], ((0, 3), (0, 0)))
  w_o1 = jnp.pad(W_o[:afd], ((0, HID - afd), (0, 0)))
  w_o2 = W_o[afd:]
  b2 = b_o.reshape(1, HID)
  zeros_acc = jnp.zeros((ACC_ROWS, HID), jnp.float32)
  zeros2 = jnp.zeros((ACC2_ROWS, HID // NC), jnp.float32)

  node_alpha = _sc_segsum(None, tgt32, zeros_acc,
                          gather_table=tree_edge_m, gidx=eids32, kmac=tree_k)
  ew = _tc_matmul(ea8, w_i2)
  xw, node_b = _tc_nodeprep(xpad, node_alpha, w_i1, W_h)
  g1 = _sc_gather(xw, src32)
  g2 = _sc_gather(node_b, src32)
  msg, bmat = _tc_combine(g1, g2, ew)
  for _ in range(depth - 1):
    in_sum = _sc_segsum_cols(msg, dst32, zeros2)
    g = _sc_gather(in_sum, src32)
    msg = _tc_update(bmat, g, msg, W_h)
  m = _sc_segsum_cols(msg, dst32, zeros2)
  return _tc_final(xpad, m, node_alpha, gids, w_o1, w_o2, b2, n_graphs)
